# baseline (speedup 1.0000x reference)
import functools

import jax
import jax.numpy as jnp
from jax import lax
from jax.experimental import pallas as pl
from jax.experimental.pallas import tpu as pltpu
from jax.experimental.pallas import tpu_sc as plsc

_NUM_EXPERTS = 64
_TOP_K = 8
_ALPHA = 0.01
_GAMMA = 0.001
_ROWS = 32768
_LANES = 128
_EH = _NUM_EXPERTS // 8
_RH = _ROWS // _LANES

_NC = 2
_NS = 16
_NW = _NC * _NS
_L = 16
_RPW = _ROWS // _NW
_RHW = _RPW // _LANES
_G = 2
_SB = _G * _L
_NSB = _RPW // _SB

_SORT8 = [(0, 1), (2, 3), (4, 5), (6, 7),
          (0, 2), (1, 3), (4, 6), (5, 7),
          (1, 2), (5, 6),
          (0, 4), (1, 5), (2, 6), (3, 7),
          (2, 4), (3, 5),
          (1, 2), (3, 4), (5, 6)]
_BITONIC8 = [(0, 4), (1, 5), (2, 6), (3, 7),
             (0, 2), (1, 3), (4, 6), (5, 7),
             (0, 1), (2, 3), (4, 5), (6, 7)]


def _sc_body(x_hbm, out_hbm, xbuf, cntbuf, thrbuf, sem):
    c = lax.axis_index("c")
    s = lax.axis_index("s")
    wid = s * _NC + c
    rhi0 = wid * _RHW

    pltpu.async_copy(x_hbm.at[:, pl.ds(rhi0, _RHW)], xbuf, sem).wait()

    iota = lax.broadcasted_iota(jnp.int32, (_L,), 0)
    neginf = jnp.full((_L,), -jnp.inf, jnp.float32)
    zeros = jnp.zeros((_L,), jnp.float32)

    def superblock(sb, cacc):
        rhi = sb // (_LANES // _SB)
        rlo0 = (sb % (_LANES // _SB)) * _SB

        def batch(it, r):
            r = list(r)
            eh = it
            for g in range(_G):
                b = [xbuf[eh, rhi, k, pl.ds(rlo0 + g * _L, _L)]
                     for k in range(_TOP_K)]
                for i, j in _SORT8:
                    lo = jnp.minimum(b[i], b[j])
                    b[j] = jnp.maximum(b[i], b[j])
                    b[i] = lo
                m = [jnp.maximum(r[g * _TOP_K + i], b[i])
                     for i in range(_TOP_K)]
                for i, j in _BITONIC8:
                    hi = jnp.maximum(m[i], m[j])
                    m[j] = jnp.minimum(m[i], m[j])
                    m[i] = hi
                for i in range(_TOP_K):
                    r[g * _TOP_K + i] = m[i]
            return tuple(r)

        rfin = lax.fori_loop(0, _TOP_K, batch,
                             tuple([neginf] * (_G * _TOP_K)))
        thr = [rfin[g * _TOP_K + _TOP_K - 1] for g in range(_G)]

        def chunk(eq, cacc):
            accl = jnp.zeros((_L,), jnp.float32)
            for k in range(_L):
                pc = None
                for g in range(_G):
                    v = xbuf[2 * eq + k // 8, rhi, k % 8,
                             pl.ds(rlo0 + g * _L, _L)]
                    p = plsc.all_reduce_population_count(v >= thr[g])
                    pc = p if pc is None else pc + p
                onehot = (iota == k).astype(jnp.float32)
                accl = accl + onehot * pc.astype(jnp.float32)
            return tuple(
                jnp.where(eq == j, cacc[j] + accl, cacc[j])
                for j in range(_NUM_EXPERTS // _L))

        return lax.fori_loop(0, _NUM_EXPERTS // _L, chunk, tuple(cacc))

    cacc = lax.fori_loop(0, _NSB, superblock, tuple([zeros] * 4))
    for j in range(4):
        cntbuf[pl.ds(j * _L, _L)] = cacc[j]
    pltpu.sync_copy(cntbuf, out_hbm.at[wid])


@functools.partial(
    pl.kernel,
    out_type=jax.ShapeDtypeStruct((_NW, _NUM_EXPERTS), jnp.float32),
    mesh=plsc.VectorSubcoreMesh(core_axis_name="c", subcore_axis_name="s"),
    scratch_types=[
        pltpu.VMEM((_EH, _RHW, 8, _LANES), jnp.float32),
        pltpu.VMEM((_NUM_EXPERTS,), jnp.float32),
        pltpu.VMEM((_SB,), jnp.float32),
        pltpu.SemaphoreType.DMA,
    ],
    compiler_params=pltpu.CompilerParams(needs_layout_passes=False,
                                         use_tc_tiling_on_sc=False),
)
def _sc_counts(x_hbm, out_hbm, xbuf, cntbuf, thrbuf, sem):
    _sc_body(x_hbm, out_hbm, xbuf, cntbuf, thrbuf, sem)


_TCB = 4096


def _tc_body(xt_hbm, acc_ref, buf, sem):
    pi = pl.program_id(0)
    nb = pl.num_programs(0)

    @pl.when(pi == 0)
    def _init():
        acc_ref[...] = jnp.zeros_like(acc_ref)
        pltpu.make_async_copy(
            xt_hbm.at[:, pl.ds(0, _TCB)], buf.at[0], sem.at[0]).start()

    @pl.when(pi + 1 < nb)
    def _prefetch():
        pltpu.make_async_copy(
            xt_hbm.at[:, pl.ds((pi + 1) * _TCB, _TCB)],
            buf.at[(pi + 1) % 2], sem.at[(pi + 1) % 2]).start()

    pltpu.make_async_copy(
        xt_hbm.at[:, pl.ds(pi * _TCB, _TCB)], buf.at[pi % 2], sem.at[pi % 2]
    ).wait()

    x = buf[pi % 2]
    m = jnp.max(x, axis=0, keepdims=True)
    ex = jnp.exp(x - m)
    s = jnp.sum(ex, axis=0, keepdims=True)
    lse = m + jnp.log(s)
    z_part = jnp.sum(lse * lse, axis=1, keepdims=True)
    prob_part = jnp.sum(ex / s, axis=1, keepdims=True)

    acc_ref[0:_NUM_EXPERTS, 0:1] += prob_part
    acc_ref[0:1, 1:2] += z_part


def _tc_softmax_stats(xt):
    xt = pltpu.with_memory_space_constraint(xt, pltpu.MemorySpace.HBM)
    return pl.pallas_call(
        _tc_body,
        grid=(_ROWS // _TCB,),
        in_specs=[pl.BlockSpec(memory_space=pl.ANY)],
        out_specs=pl.BlockSpec((_NUM_EXPERTS, 128), lambda i: (0, 0)),
        out_shape=jax.ShapeDtypeStruct((_NUM_EXPERTS, 128), jnp.float32),
        scratch_shapes=[
            pltpu.VMEM((2, _NUM_EXPERTS, _TCB), jnp.float32),
            pltpu.SemaphoreType.DMA((2,)),
        ],
    )(xt)


@jax.jit
def kernel(router_logits):
    xt = router_logits.T
    xv = xt.reshape(_EH, 8, _RH, _LANES).transpose(0, 2, 1, 3)
    cnt_parts = _sc_counts(xv)
    acc = _tc_softmax_stats(xt)
    inv_n = 1.0 / _ROWS
    expert_prob = acc[:, 0] * inv_n
    expert_freq = jnp.sum(cnt_parts, axis=0) * inv_n
    z_loss = acc[0, 1] * inv_n
    global_loss = _NUM_EXPERTS * jnp.sum(expert_prob * expert_freq)
    return _ALPHA * global_loss + _GAMMA * z_loss
